# baseline (speedup 1.0000x reference)
import functools

import jax
import jax.numpy as jnp
from jax import lax
from jax.experimental import pallas as pl
from jax.experimental.pallas import tpu as pltpu
from jax.experimental.pallas import tpu_sc as plsc

B = 16384
F = 26
V = 1000
D = 50
DP = 56
N = B * F
SPLIT = 2
NSPL = N // SPLIT

NC = 2
NS = 16
NW = NC * NS
NPW = NSPL // NW
STREAM = 104
NSTREAM = 4
SC_CHUNK = NSTREAM * STREAM
NG = NPW // SC_CHUNK


def _body(idx_hbm, tab_hbm, out_hbm, idx_v, rows0, rows1, sem0, sem1):
    wid = lax.axis_index("s") * NC + lax.axis_index("c")
    base = wid * NPW

    pltpu.sync_copy(idx_hbm.at[pl.ds(base, NPW)], idx_v)

    def copies(g, rows, sem):
        s0 = pl.multiple_of(g * SC_CHUNK, SC_CHUNK)
        return [
            pltpu.make_async_copy(
                tab_hbm.at[idx_v.at[pl.ds(s0 + j * STREAM, STREAM)]],
                rows.at[pl.ds(j * STREAM, STREAM)], sem)
            for j in range(NSTREAM)
        ]

    def fire(g, rows, sem):
        for cp in copies(g, rows, sem):
            cp.start()

    def drain(g, rows, sem):
        for cp in copies(g, rows, sem):
            cp.wait()

    def store(g, rows):
        s0 = pl.multiple_of(g * SC_CHUNK, SC_CHUNK)
        pltpu.sync_copy(rows, out_hbm.at[pl.ds(base + s0, SC_CHUNK)])

    fire(0, rows0, sem0)

    def g_body(k, carry):
        g0 = pl.multiple_of(2 * k, 2)
        fire(g0 + 1, rows1, sem1)
        drain(g0, rows0, sem0)
        store(g0, rows0)

        @pl.when(g0 + 2 < NG)
        def _():
            fire(g0 + 2, rows0, sem0)

        drain(g0 + 1, rows1, sem1)
        store(g0 + 1, rows1)
        return carry

    lax.fori_loop(0, NG // 2, g_body, 0)


@functools.partial(
    pl.kernel,
    out_type=jax.ShapeDtypeStruct((NSPL, DP), jnp.float32),
    mesh=plsc.VectorSubcoreMesh(core_axis_name="c", subcore_axis_name="s"),
    compiler_params=pltpu.CompilerParams(use_tc_tiling_on_sc=False),
    scratch_types=[
        pltpu.VMEM((NPW,), jnp.int32),
        pltpu.VMEM((SC_CHUNK, DP), jnp.float32),
        pltpu.VMEM((SC_CHUNK, DP), jnp.float32),
        pltpu.SemaphoreType.DMA,
        pltpu.SemaphoreType.DMA,
    ],
)
def _gather_kernel(idx_hbm, tab_hbm, out_hbm, idx_v, rows0, rows1, sem0, sem1):
    _body(idx_hbm, tab_hbm, out_hbm, idx_v, rows0, rows1, sem0, sem1)


def kernel(indices, tables):
    idx_flat = (indices.astype(jnp.int32)
                + jnp.arange(F, dtype=jnp.int32)[None, :] * V).reshape(N)
    tab_pad = jnp.pad(tables.reshape(F * V, D), ((0, 0), (0, DP - D)))
    halves = [
        _gather_kernel(
            jax.lax.slice(idx_flat, (h * NSPL,), ((h + 1) * NSPL,)), tab_pad
        ).reshape(B // SPLIT, F * DP)
        for h in range(SPLIT)
    ]
    out2 = jnp.concatenate(halves, axis=0)
    cols = (jnp.arange(F * D, dtype=jnp.int32) // D) * DP + (
        jnp.arange(F * D, dtype=jnp.int32) % D)
    return jnp.take(out2, cols, axis=1, mode="clip")


# speedup vs baseline: 1.0635x; 1.0635x over previous
import functools

import jax
import jax.numpy as jnp
from jax import lax
from jax.experimental import pallas as pl
from jax.experimental.pallas import tpu as pltpu
from jax.experimental.pallas import tpu_sc as plsc

B = 16384
F = 26
V = 1000
D = 50
DP = 56
N = B * F
SPLIT = 1
NSPL = N // SPLIT

NC = 2
NS = 16
NW = NC * NS
NPW = NSPL // NW
STREAM = 104
NSTREAM = 4
SC_CHUNK = NSTREAM * STREAM
NG = NPW // SC_CHUNK


def _body(idx_hbm, tab_hbm, out_hbm, idx_v, rows0, rows1, sem0, sem1):
    wid = lax.axis_index("s") * NC + lax.axis_index("c")
    base = wid * NPW

    pltpu.sync_copy(idx_hbm.at[pl.ds(base, NPW)], idx_v)

    def copies(g, rows, sem):
        s0 = pl.multiple_of(g * SC_CHUNK, SC_CHUNK)
        return [
            pltpu.make_async_copy(
                tab_hbm.at[idx_v.at[pl.ds(s0 + j * STREAM, STREAM)]],
                rows.at[pl.ds(j * STREAM, STREAM)], sem)
            for j in range(NSTREAM)
        ]

    def fire(g, rows, sem):
        for cp in copies(g, rows, sem):
            cp.start()

    def drain(g, rows, sem):
        for cp in copies(g, rows, sem):
            cp.wait()

    def store(g, rows):
        s0 = pl.multiple_of(g * SC_CHUNK, SC_CHUNK)
        pltpu.sync_copy(rows, out_hbm.at[pl.ds(base + s0, SC_CHUNK)])

    fire(0, rows0, sem0)

    def g_body(k, carry):
        g0 = pl.multiple_of(2 * k, 2)
        fire(g0 + 1, rows1, sem1)
        drain(g0, rows0, sem0)
        store(g0, rows0)

        @pl.when(g0 + 2 < NG)
        def _():
            fire(g0 + 2, rows0, sem0)

        drain(g0 + 1, rows1, sem1)
        store(g0 + 1, rows1)
        return carry

    lax.fori_loop(0, NG // 2, g_body, 0)


@functools.partial(
    pl.kernel,
    out_type=jax.ShapeDtypeStruct((NSPL, DP), jnp.float32),
    mesh=plsc.VectorSubcoreMesh(core_axis_name="c", subcore_axis_name="s"),
    compiler_params=pltpu.CompilerParams(use_tc_tiling_on_sc=False),
    scratch_types=[
        pltpu.VMEM((NPW,), jnp.int32),
        pltpu.VMEM((SC_CHUNK, DP), jnp.float32),
        pltpu.VMEM((SC_CHUNK, DP), jnp.float32),
        pltpu.SemaphoreType.DMA,
        pltpu.SemaphoreType.DMA,
    ],
)
def _gather_kernel(idx_hbm, tab_hbm, out_hbm, idx_v, rows0, rows1, sem0, sem1):
    _body(idx_hbm, tab_hbm, out_hbm, idx_v, rows0, rows1, sem0, sem1)


def kernel(indices, tables):
    idx_flat = (indices.astype(jnp.int32).reshape(N)
                + (jnp.arange(N, dtype=jnp.int32) % F) * V)
    tab_pad = jnp.pad(tables.reshape(F * V, D), ((0, 0), (0, DP - D)))
    halves = [
        _gather_kernel(
            jax.lax.slice(idx_flat, (h * NSPL,), ((h + 1) * NSPL,)), tab_pad
        ).reshape(B // SPLIT, F * DP)
        for h in range(SPLIT)
    ]
    out2 = jnp.concatenate(halves, axis=0)
    cols = (jnp.arange(F * D, dtype=jnp.int32) // D) * DP + (
        jnp.arange(F * D, dtype=jnp.int32) % D)
    return jnp.take(out2, cols, axis=1, mode="clip")
